# baseline (speedup 1.0000x reference)
import jax
import jax.numpy as jnp
from jax.experimental import pallas as pl
from jax.experimental.pallas import tpu as pltpu

_CH = 1024


def _body(nn_ref, q_ref, m_ref, o_ref, mbuf, sem):
    b = pl.program_id(0)
    nn = nn_ref[b]
    nc = pl.cdiv(nn, _CH)
    neg = jnp.finfo(jnp.float32).min
    n_total = m_ref.shape[1]
    n_chunks = pl.cdiv(n_total, _CH)

    def copy(k, size):
        slot = k % 2
        return pltpu.make_async_copy(
            m_ref.at[b, pl.ds(k * _CH, size), :],
            mbuf.at[slot, pl.ds(0, size), :],
            sem.at[slot],
        )

    first_size = min(_CH, n_total)
    copy(0, first_size).start()
    q = q_ref[0]

    for k in range(n_chunks):
        start = k * _CH
        size = min(_CH, n_total - start)
        slot = k % 2

        @pl.when(k < nc)
        def _valid():
            if k + 1 < n_chunks:
                next_size = min(_CH, n_total - (k + 1) * _CH)

                @pl.when(k + 1 < nc)
                def _prefetch():
                    copy(k + 1, next_size).start()

            copy(k, size).wait()
            m = mbuf[slot, :size]
            logits = jax.lax.dot_general(
                q, m, (((1,), (1,)), ((), ())),
                preferred_element_type=jnp.float32,
            )
            n_idx = start + jax.lax.broadcasted_iota(jnp.int32, logits.shape, 1)
            o_ref[0, :, start:start + size] = jnp.where(n_idx < nn, logits, neg)

        @pl.when(k >= nc)
        def _fill():
            o_ref[0, :, start:start + size] = jnp.full(
                (q_ref.shape[1], size), neg, jnp.float32
            )


def kernel(query_vector, node_nums, m_A):
    B, Q, D = query_vector.shape
    N = m_A.shape[1]

    grid_spec = pltpu.PrefetchScalarGridSpec(
        num_scalar_prefetch=1,
        grid=(B,),
        in_specs=[
            pl.BlockSpec((1, Q, D), lambda b, nn_ref: (b, 0, 0)),
            pl.BlockSpec(memory_space=pltpu.MemorySpace.HBM),
        ],
        out_specs=pl.BlockSpec((1, Q, N), lambda b, nn_ref: (b, 0, 0)),
        scratch_shapes=[
            pltpu.VMEM((2, _CH, D), jnp.float32),
            pltpu.SemaphoreType.DMA((2,)),
        ],
    )
    return pl.pallas_call(
        _body,
        grid_spec=grid_spec,
        out_shape=jax.ShapeDtypeStruct((B, Q, N), jnp.float32),
    )(node_nums.astype(jnp.int32), query_vector, m_A)


# speedup vs baseline: 1.7064x; 1.7064x over previous
import jax
import jax.numpy as jnp
from jax.experimental import pallas as pl
from jax.experimental.pallas import tpu as pltpu

_CH = 1024


def _body(nn_ref, q_ref, m_ref, o_ref, mbuf, sem):
    b = pl.program_id(0)
    nn = nn_ref[b]
    nc = pl.cdiv(nn, _CH)
    neg = jnp.finfo(jnp.float32).min
    n_total = m_ref.shape[1]
    n_chunks = pl.cdiv(n_total, _CH)

    def copy(k, size):
        return pltpu.make_async_copy(
            m_ref.at[b, pl.ds(k * _CH, size), :],
            mbuf.at[k, pl.ds(0, size), :],
            sem.at[k],
        )

    for k in range(n_chunks):
        size = min(_CH, n_total - k * _CH)
        if k == 0:
            copy(k, size).start()
        else:
            @pl.when(k < nc)
            def _start():
                copy(k, size).start()

    q = q_ref[0]

    for k in range(n_chunks):
        start = k * _CH
        size = min(_CH, n_total - start)

        @pl.when(k < nc)
        def _valid():
            copy(k, size).wait()
            m = mbuf[k, :size]
            logits = jax.lax.dot_general(
                q, m, (((1,), (1,)), ((), ())),
                preferred_element_type=jnp.float32,
            )
            n_idx = start + jax.lax.broadcasted_iota(jnp.int32, logits.shape, 1)
            o_ref[0, :, start:start + size] = jnp.where(n_idx < nn, logits, neg)

        @pl.when(k >= nc)
        def _fill():
            o_ref[0, :, start:start + size] = jnp.full(
                (q_ref.shape[1], size), neg, jnp.float32
            )


def kernel(query_vector, node_nums, m_A):
    B, Q, D = query_vector.shape
    N = m_A.shape[1]

    grid_spec = pltpu.PrefetchScalarGridSpec(
        num_scalar_prefetch=1,
        grid=(B,),
        in_specs=[
            pl.BlockSpec((1, Q, D), lambda b, nn_ref: (b, 0, 0)),
            pl.BlockSpec(memory_space=pltpu.MemorySpace.HBM),
        ],
        out_specs=pl.BlockSpec((1, Q, N), lambda b, nn_ref: (b, 0, 0)),
        scratch_shapes=[
            pltpu.VMEM((pl.cdiv(N, _CH), _CH, D), jnp.float32),
            pltpu.SemaphoreType.DMA((pl.cdiv(N, _CH),)),
        ],
    )
    return pl.pallas_call(
        _body,
        grid_spec=grid_spec,
        out_shape=jax.ShapeDtypeStruct((B, Q, N), jnp.float32),
    )(node_nums.astype(jnp.int32), query_vector, m_A)


# speedup vs baseline: 2.5104x; 1.4712x over previous
import jax
import jax.numpy as jnp
from jax.experimental import pallas as pl
from jax.experimental.pallas import tpu as pltpu

_CH = 2048


def _body(nn_ref, q_ref, m_ref, o_ref, mbuf, sem):
    b = pl.program_id(0)
    num_b = pl.num_programs(0)
    nn = nn_ref[b]
    nc = pl.cdiv(nn, _CH)
    neg = jnp.finfo(jnp.float32).min
    n_total = m_ref.shape[1]
    n_chunks = pl.cdiv(n_total, _CH)

    def issue(bb, parity):
        ncb = pl.cdiv(nn_ref[bb], _CH)
        for k in range(n_chunks):
            size = min(_CH, n_total - k * _CH)

            @pl.when(k < ncb)
            def _start():
                pltpu.make_async_copy(
                    m_ref.at[bb, pl.ds(k * _CH, size), :],
                    mbuf.at[parity, k, pl.ds(0, size), :],
                    sem.at[parity, k],
                ).start()

    @pl.when(b == 0)
    def _prologue():
        issue(0, 0)

    @pl.when(b + 1 < num_b)
    def _prefetch_next():
        issue(b + 1, (b + 1) % 2)

    q = q_ref[0]
    par = b % 2

    for k in range(n_chunks):
        start = k * _CH
        size = min(_CH, n_total - start)

        @pl.when(k < nc)
        def _valid():
            pltpu.make_async_copy(
                m_ref.at[b, pl.ds(start, size), :],
                mbuf.at[par, k, pl.ds(0, size), :],
                sem.at[par, k],
            ).wait()
            m = mbuf[par, k, :size]
            logits = jax.lax.dot_general(
                q, m, (((1,), (1,)), ((), ())),
                preferred_element_type=jnp.float32,
            )
            n_idx = start + jax.lax.broadcasted_iota(jnp.int32, logits.shape, 1)
            o_ref[0, :, start:start + size] = jnp.where(n_idx < nn, logits, neg)

        @pl.when(k >= nc)
        def _fill():
            o_ref[0, :, start:start + size] = jnp.full(
                (q_ref.shape[1], size), neg, jnp.float32
            )


def kernel(query_vector, node_nums, m_A):
    B, Q, D = query_vector.shape
    N = m_A.shape[1]
    n_chunks = pl.cdiv(N, _CH)

    grid_spec = pltpu.PrefetchScalarGridSpec(
        num_scalar_prefetch=1,
        grid=(B,),
        in_specs=[
            pl.BlockSpec((1, Q, D), lambda b, nn_ref: (b, 0, 0)),
            pl.BlockSpec(memory_space=pltpu.MemorySpace.HBM),
        ],
        out_specs=pl.BlockSpec((1, Q, N), lambda b, nn_ref: (b, 0, 0)),
        scratch_shapes=[
            pltpu.VMEM((2, n_chunks, _CH, D), jnp.float32),
            pltpu.SemaphoreType.DMA((2, n_chunks)),
        ],
    )
    return pl.pallas_call(
        _body,
        grid_spec=grid_spec,
        out_shape=jax.ShapeDtypeStruct((B, Q, N), jnp.float32),
    )(node_nums.astype(jnp.int32), query_vector, m_A)


# speedup vs baseline: 2.5409x; 1.0121x over previous
import jax
import jax.numpy as jnp
from jax.experimental import pallas as pl
from jax.experimental.pallas import tpu as pltpu

_CH = 2048


def _body(nn_ref, q_ref, m_ref, o_ref, mbuf, sem):
    b = pl.program_id(0)
    num_b = pl.num_programs(0)
    nn = nn_ref[b]
    nc = pl.cdiv(nn, _CH)
    neg = jnp.finfo(jnp.float32).min
    n_total = m_ref.shape[1]
    n_chunks = pl.cdiv(n_total, _CH)

    def issue(bb, parity):
        ncb = pl.cdiv(nn_ref[bb], _CH)
        for k in range(n_chunks):
            size = min(_CH, n_total - k * _CH)

            @pl.when(k < ncb)
            def _start():
                pltpu.make_async_copy(
                    m_ref.at[bb, pl.ds(k * _CH, size), :],
                    mbuf.at[parity, k, pl.ds(0, size), :],
                    sem.at[parity, k],
                ).start()

    @pl.when(b == 0)
    def _prologue():
        issue(0, 0)

    @pl.when(b + 1 < num_b)
    def _prefetch_next():
        issue(b + 1, (b + 1) % 2)

    q = q_ref[0].astype(jnp.bfloat16)
    par = b % 2

    for k in range(n_chunks):
        start = k * _CH
        size = min(_CH, n_total - start)

        @pl.when(k < nc)
        def _valid():
            pltpu.make_async_copy(
                m_ref.at[b, pl.ds(start, size), :],
                mbuf.at[par, k, pl.ds(0, size), :],
                sem.at[par, k],
            ).wait()
            m = mbuf[par, k, :size].astype(jnp.bfloat16)
            logits = jax.lax.dot_general(
                q, m, (((1,), (1,)), ((), ())),
                preferred_element_type=jnp.float32,
            )
            n_idx = start + jax.lax.broadcasted_iota(jnp.int32, logits.shape, 1)
            o_ref[0, :, start:start + size] = jnp.where(n_idx < nn, logits, neg)

        @pl.when(k >= nc)
        def _fill():
            o_ref[0, :, start:start + size] = jnp.full(
                (q_ref.shape[1], size), neg, jnp.float32
            )


def kernel(query_vector, node_nums, m_A):
    B, Q, D = query_vector.shape
    N = m_A.shape[1]
    n_chunks = pl.cdiv(N, _CH)

    grid_spec = pltpu.PrefetchScalarGridSpec(
        num_scalar_prefetch=1,
        grid=(B,),
        in_specs=[
            pl.BlockSpec((1, Q, D), lambda b, nn_ref: (b, 0, 0)),
            pl.BlockSpec(memory_space=pltpu.MemorySpace.HBM),
        ],
        out_specs=pl.BlockSpec((1, Q, N), lambda b, nn_ref: (b, 0, 0)),
        scratch_shapes=[
            pltpu.VMEM((2, n_chunks, _CH, D), jnp.float32),
            pltpu.SemaphoreType.DMA((2, n_chunks)),
        ],
    )
    return pl.pallas_call(
        _body,
        grid_spec=grid_spec,
        out_shape=jax.ShapeDtypeStruct((B, Q, N), jnp.float32),
    )(node_nums.astype(jnp.int32), query_vector, m_A)
